# baseline (speedup 1.0000x reference)
import jax
import jax.numpy as jnp
from jax import lax
from jax.experimental import pallas as pl
from jax.experimental.pallas import tpu as pltpu
from jax.experimental.pallas import tpu_sc as plsc

NWORKERS = 32
CH = 1280
INT32_MAX = 2**31 - 1


def _body(ctx_hbm, tab_hbm, out_hbm, idx_v, rows0, rows1,
          isem, gsem0, gsem1, ssem0, ssem1):
    B = ctx_hbm.shape[0]
    w32 = tab_hbm.shape[1]
    per_w = B // NWORKERS
    nchunk = per_w // CH
    wid = lax.axis_index("s") * 2 + lax.axis_index("c")
    base = wid * per_w

    rows = (rows0, rows1)
    gsem = (gsem0, gsem1)
    ssem = (ssem0, ssem1)

    pltpu.async_copy(ctx_hbm.at[pl.ds(base, per_w)], idx_v, isem).wait()

    def start_gather(k):
        s = k & 1
        return pltpu.async_copy(
            tab_hbm.at[idx_v.at[pl.ds(k * CH, CH)]], rows[s], gsem[s])

    def chunk_has_pad(k):
        def step(g, m):
            return m | (idx_v[pl.ds(k * CH + g * 16, 16)] == 0)
        init = jnp.zeros((16,), jnp.bool_)
        return jnp.any(lax.fori_loop(0, CH // 16, step, init))

    zeros16 = jnp.zeros((16,), jnp.int32)

    def fix_chunk(k):
        s = k & 1

        def fix_group(g, carry):
            iv = idx_v[pl.ds(k * CH + g * 16, 16)]
            m = iv == 0
            rid = g * 16 + lax.iota(jnp.int32, 16)
            for w in range(w32):
                plsc.store_scatter(
                    rows[s], [rid, jnp.full((16,), w, jnp.int32)], zeros16, mask=m)
            return carry

        lax.fori_loop(0, CH // 16, fix_group, 0)

    gdesc = [None, None]
    sdesc = [None, None]
    gdesc[0] = start_gather(0)
    for k in range(nchunk):
        s = k & 1
        if k + 1 < nchunk:
            if sdesc[1 - s] is not None:
                sdesc[1 - s].wait()
                sdesc[1 - s] = None
            gdesc[1 - s] = start_gather(k + 1)
        haspad = chunk_has_pad(k)
        gdesc[s].wait()

        @pl.when(haspad)
        def _():
            fix_chunk(k)

        sdesc[s] = pltpu.async_copy(
            rows[s], out_hbm.at[pl.ds(base + k * CH, CH)], ssem[s])
    for d in sdesc:
        if d is not None:
            d.wait()


def kernel(context, table):
    V, D = table.shape
    B = context.size
    w32 = D // 2
    ctx = context.reshape(B)
    tab_i32 = lax.bitcast_convert_type(table.reshape(V, w32, 2), jnp.int32)

    call = pl.kernel(
        _body,
        out_type=jax.ShapeDtypeStruct((B, w32), jnp.int32),
        mesh=plsc.VectorSubcoreMesh(core_axis_name="c", subcore_axis_name="s"),
        compiler_params=pltpu.CompilerParams(use_tc_tiling_on_sc=False, needs_layout_passes=False),
        scratch_types=[
            pltpu.VMEM((B // NWORKERS,), jnp.int32),
            pltpu.VMEM((CH, w32), jnp.int32),
            pltpu.VMEM((CH, w32), jnp.int32),
            pltpu.SemaphoreType.DMA,
            pltpu.SemaphoreType.DMA,
            pltpu.SemaphoreType.DMA,
            pltpu.SemaphoreType.DMA,
            pltpu.SemaphoreType.DMA,
        ],
    )
    out = call(ctx, tab_i32)
    return lax.bitcast_convert_type(out, jnp.bfloat16).reshape(
        context.shape + (D,))


# speedup vs baseline: 2.4020x; 2.4020x over previous
import jax
import jax.numpy as jnp
from jax import lax
from jax.experimental import pallas as pl
from jax.experimental.pallas import tpu as pltpu
from jax.experimental.pallas import tpu_sc as plsc

NWORKERS = 32
CH = 1280


def _body(ctx_hbm, tab_hbm, out_hbm, idx_v, rows0, rows1,
          isem, gsem0, gsem1, ssem0, ssem1):
    B = ctx_hbm.shape[0]
    D = tab_hbm.shape[1]
    per_w = B // NWORKERS
    nchunk = per_w // CH
    wid = lax.axis_index("s") * 2 + lax.axis_index("c")
    base = wid * per_w

    rows = (rows0, rows1)
    gsem = (gsem0, gsem1)
    ssem = (ssem0, ssem1)

    pltpu.async_copy(ctx_hbm.at[pl.ds(base, per_w)], idx_v, isem).wait()

    def start_gather(k):
        s = k & 1
        return pltpu.async_copy(
            tab_hbm.at[idx_v.at[pl.ds(k * CH, CH)]], rows[s], gsem[s])

    def chunk_has_pad(k):
        def step(g, m):
            return m | (idx_v[pl.ds(k * CH + g * 16, 16)] == 0)
        init = jnp.zeros((16,), jnp.bool_)
        return jnp.any(lax.fori_loop(0, CH // 16, step, init))

    zrow = jnp.zeros((32,), jnp.bfloat16)

    def fix_chunk(k):
        s = k & 1

        def fix_group(g, carry):
            iv = idx_v[pl.ds(k * CH + g * 16, 16)]
            for l in range(16):
                r = g * 16 + l

                @pl.when(iv[l] == 0)
                def _():
                    rows[s][r, pl.ds(0, 32)] = zrow
                    rows[s][r, pl.ds(32, 32)] = zrow

            return carry

        lax.fori_loop(0, CH // 16, fix_group, 0)

    gdesc = [None, None]
    sdesc = [None, None]
    gdesc[0] = start_gather(0)
    for k in range(nchunk):
        s = k & 1
        if k + 1 < nchunk:
            if sdesc[1 - s] is not None:
                sdesc[1 - s].wait()
                sdesc[1 - s] = None
            gdesc[1 - s] = start_gather(k + 1)
        haspad = chunk_has_pad(k)
        gdesc[s].wait()

        @pl.when(haspad)
        def _():
            fix_chunk(k)

        sdesc[s] = pltpu.async_copy(
            rows[s], out_hbm.at[pl.ds(base + k * CH, CH)], ssem[s])
    for d in sdesc:
        if d is not None:
            d.wait()


def kernel(context, table):
    V, D = table.shape
    B = context.size
    ctx = context.reshape(B)

    call = pl.kernel(
        _body,
        out_type=jax.ShapeDtypeStruct((B, D), jnp.bfloat16),
        mesh=plsc.VectorSubcoreMesh(core_axis_name="c", subcore_axis_name="s"),
        compiler_params=pltpu.CompilerParams(
            use_tc_tiling_on_sc=False, needs_layout_passes=False),
        scratch_types=[
            pltpu.VMEM((B // NWORKERS,), jnp.int32),
            pltpu.VMEM((CH, D), jnp.bfloat16),
            pltpu.VMEM((CH, D), jnp.bfloat16),
            pltpu.SemaphoreType.DMA,
            pltpu.SemaphoreType.DMA,
            pltpu.SemaphoreType.DMA,
            pltpu.SemaphoreType.DMA,
            pltpu.SemaphoreType.DMA,
        ],
    )
    out = call(ctx, table)
    return out.reshape(context.shape + (D,))
